# baseline (speedup 1.0000x reference)
import jax
import jax.numpy as jnp
from jax.experimental import pallas as pl
from jax.experimental.pallas import tpu as pltpu

N = 4096
F = 64
LAM0 = 1.0
LAM1 = 1.0
LAM4 = 0.0
ALPHA = 1.0 / (1.0 + LAM4 + LAM0 + LAM1)

BLK1 = 256
G1 = N // BLK1
BLK3 = 512
G3 = N // BLK3


def _step1_fused(ab_ref, ag_ref, db_ref, dg_ref, i_ref, xf_ref, xb_ref,
                 s_ref, qs_ref, y1_ref, y1b_ref, yhat_ref):
    p = pl.program_id(0)
    i = pl.program_id(1)
    rows = pl.ds(i * BLK1, BLK1)

    @pl.when(p == 0)
    def _():
        s = LAM0 * ab_ref[...] + LAM1 * ag_ref[...]
        s_ref[...] = s.astype(jnp.bfloat16)
        yhat_ref[rows, :] = (
            jnp.dot(s, xf_ref[...], preferred_element_type=jnp.float32)
            + xb_ref[...]
        )

    @pl.when(p == 1)
    def _():
        qs = ALPHA / (LAM0 * db_ref[...] + LAM1 * dg_ref[...] + i_ref[...])
        qs_ref[...] = qs.astype(jnp.bfloat16)
        y1 = (1.0 - ALPHA) * xb_ref[...] + jnp.dot(
            qs, yhat_ref[...], preferred_element_type=jnp.float32)
        y1_ref[rows, :] = y1
        y1b_ref[rows, :] = y1.astype(jnp.bfloat16)


def _step2_fused(s_ref, qs_ref, y1b_ref, y1_ref, x_ref, out_ref, yhat_ref):
    p = pl.program_id(0)
    i = pl.program_id(1)
    rows = pl.ds(i * BLK3, BLK3)

    @pl.when(p == 0)
    def _():
        yhat = (
            jnp.dot(s_ref[...], y1b_ref[...],
                    preferred_element_type=jnp.float32)
            + x_ref[...]
        )
        yhat_ref[rows, :] = yhat.astype(jnp.bfloat16)

    @pl.when(p == 1)
    def _():
        out_ref[...] = (1.0 - ALPHA) * y1_ref[...] + jnp.dot(
            qs_ref[...], yhat_ref[...], preferred_element_type=jnp.float32)


def kernel(X, A_beta, A_gamma, D_beta, D_gamma, I_mat):
    f32 = jnp.float32
    bf16 = jnp.bfloat16

    _sx, _qx, y1, y1b = pl.pallas_call(
        _step1_fused,
        grid=(2, G1),
        in_specs=[
            pl.BlockSpec((BLK1, N), lambda p, i: (i * (1 - p) + (G1 - 1) * p, 0)),
            pl.BlockSpec((BLK1, N), lambda p, i: (i * (1 - p) + (G1 - 1) * p, 0)),
            pl.BlockSpec((BLK1, N), lambda p, i: (i * p, 0)),
            pl.BlockSpec((BLK1, N), lambda p, i: (i * p, 0)),
            pl.BlockSpec((BLK1, N), lambda p, i: (i * p, 0)),
            pl.BlockSpec((N, F), lambda p, i: (0, 0)),
            pl.BlockSpec((BLK1, F), lambda p, i: (i, 0)),
        ],
        out_specs=(
            pl.BlockSpec((BLK1, N), lambda p, i: (i * (1 - p) + G1 * p, 0)),
            pl.BlockSpec((BLK1, N), lambda p, i: (i * p + G1 * (1 - p), 0)),
            pl.BlockSpec((N, F), lambda p, i: (0, 0)),
            pl.BlockSpec((N, F), lambda p, i: (0, 0)),
        ),
        out_shape=(
            jax.ShapeDtypeStruct((N + BLK1, N), bf16),
            jax.ShapeDtypeStruct((N + BLK1, N), bf16),
            jax.ShapeDtypeStruct((N, F), f32),
            jax.ShapeDtypeStruct((N, F), bf16),
        ),
        scratch_shapes=[pltpu.VMEM((N, F), f32)],
        compiler_params=pltpu.CompilerParams(
            dimension_semantics=("arbitrary", "arbitrary")),
    )(A_beta, A_gamma, D_beta, D_gamma, I_mat, X, X)

    y2 = pl.pallas_call(
        _step2_fused,
        grid=(2, G3),
        in_specs=[
            pl.BlockSpec((BLK3, N), lambda p, i: (i * (1 - p) + (G3 - 1) * p, 0)),
            pl.BlockSpec((BLK3, N), lambda p, i: (i * p, 0)),
            pl.BlockSpec((N, F), lambda p, i: (0, 0)),
            pl.BlockSpec((BLK3, F), lambda p, i: (i, 0)),
            pl.BlockSpec((BLK3, F), lambda p, i: (i, 0)),
        ],
        out_specs=pl.BlockSpec((BLK3, F), lambda p, i: (i, 0)),
        out_shape=jax.ShapeDtypeStruct((N, F), f32),
        scratch_shapes=[pltpu.VMEM((N, F), bf16)],
        compiler_params=pltpu.CompilerParams(
            dimension_semantics=("arbitrary", "arbitrary")),
    )(_sx, _qx, y1b, y1, X)

    return y2
